# baseline (speedup 1.0000x reference)
import functools

import jax
import jax.numpy as jnp
from jax import lax
from jax.experimental import pallas as pl
from jax.experimental.pallas import tpu as pltpu
from jax.experimental.pallas import tpu_sc as plsc


@functools.lru_cache(maxsize=None)
def _build_gather(V, D, E, chunk):
    info = plsc.get_sparse_core_info()
    NC, NS = info.num_cores, info.num_subcores
    NW = NC * NS
    assert E % NW == 0
    e_per_w = E // NW
    assert e_per_w % chunk == 0 and chunk % 8 == 0
    nchunks = e_per_w // chunk
    assert nchunks % 2 == 1 and nchunks >= 3
    npairs = nchunks // 2
    last = nchunks - 1
    v_per_s = (V // NS) // 8 * 8
    v_last = V - v_per_s * (NS - 1)
    mesh = plsc.VectorSubcoreMesh(core_axis_name="c", subcore_axis_name="s")

    @functools.partial(
        pl.kernel,
        mesh=mesh,
        out_type=jax.ShapeDtypeStruct((E, 2 * D), jnp.float32),
        scratch_types=[
            pltpu.VMEM_SHARED((V, D), jnp.float32),
            pltpu.VMEM((chunk,), jnp.int32),
            pltpu.VMEM((chunk,), jnp.int32),
            pltpu.VMEM((chunk,), jnp.int32),
            pltpu.VMEM((chunk,), jnp.int32),
            pltpu.VMEM((chunk, D), jnp.float32),
            pltpu.VMEM((chunk, D), jnp.float32),
            pltpu.VMEM((chunk, D), jnp.float32),
            pltpu.VMEM((chunk, D), jnp.float32),
            pltpu.SemaphoreType.DMA,
            pltpu.SemaphoreType.DMA,
            pltpu.SemaphoreType.DMA,
            pltpu.SemaphoreType.DMA,
            pltpu.SemaphoreType.DMA,
            pltpu.SemaphoreType.DMA,
            pltpu.SemaphoreType.DMA,
            pltpu.SemaphoreType.DMA,
            pltpu.SemaphoreType.DMA,
            pltpu.SemaphoreType.DMA,
            pltpu.SemaphoreType.DMA,
            pltpu.SemaphoreType.DMA,
        ],
    )
    def gather_kernel(
        table_hbm, idx_hbm, out_hbm,
        table_s, sidx_a, tidx_a, sidx_b, tidx_b,
        srows_a, trows_a, srows_b, trows_b,
        iss_a, ist_a, iss_b, ist_b,
        gss_a, gst_a, gss_b, gst_b,
        oss_a, ost_a, oss_b, ost_b,
    ):
        sub = lax.axis_index("s")
        wid = sub * NC + lax.axis_index("c")
        base = wid * e_per_w

        row0 = sub * v_per_s
        nrow = jnp.where(sub == NS - 1, v_last, v_per_s)
        pltpu.sync_copy(
            table_hbm.at[pl.ds(row0, nrow)], table_s.at[pl.ds(row0, nrow)]
        )
        plsc.subcore_barrier()

        def i_copies(g, sbuf, tbuf, ssem, tsem):
            off = base + g * chunk
            return (
                pltpu.make_async_copy(
                    idx_hbm.at[pl.ds(off, chunk)], sbuf, ssem
                ),
                pltpu.make_async_copy(
                    idx_hbm.at[pl.ds(E + off, chunk)], tbuf, tsem
                ),
            )

        def g_copies(tab, sibuf, tibuf, sbuf, tbuf, ssem, tsem):
            return (
                pltpu.make_async_copy(tab.at[sibuf], sbuf, ssem),
                pltpu.make_async_copy(tab.at[tibuf], tbuf, tsem),
            )

        def w_copies(g, sbuf, tbuf, ssem, tsem):
            orow = base + g * chunk
            return (
                pltpu.make_async_copy(
                    sbuf, out_hbm.at[pl.ds(orow, chunk), pl.ds(0, D)], ssem
                ),
                pltpu.make_async_copy(
                    tbuf, out_hbm.at[pl.ds(orow, chunk), pl.ds(D, D)], tsem
                ),
            )

        def g_a():
            return g_copies(table_s, sidx_a, tidx_a, srows_a, trows_a, gss_a, gst_a)

        def g_b():
            return g_copies(table_hbm, sidx_b, tidx_b, srows_b, trows_b, gss_b, gst_b)

        def i_a(g):
            return i_copies(g, sidx_a, tidx_a, iss_a, ist_a)

        def i_b(g):
            return i_copies(g, sidx_b, tidx_b, iss_b, ist_b)

        def start(copies):
            for c in copies:
                c.start()

        def wait(copies):
            for c in copies:
                c.wait()

        start(i_a(0))
        start(i_b(1))
        wait(i_a(0))
        start(g_a())

        def body(p, carry):
            g0 = 2 * p
            g1 = g0 + 1
            wait(i_b(g1))
            start(g_b())
            wait(g_a())
            start(i_a(g0 + 2))
            start(w_copies(g0, srows_a, trows_a, oss_a, ost_a))
            wait(g_b())

            @pl.when(p < npairs - 1)
            def _():
                start(i_b(g1 + 2))

            start(w_copies(g1, srows_b, trows_b, oss_b, ost_b))
            wait(w_copies(g0, srows_a, trows_a, oss_a, ost_a))
            wait(i_a(g0 + 2))
            start(g_a())
            wait(w_copies(g1, srows_b, trows_b, oss_b, ost_b))
            return carry

        lax.fori_loop(0, npairs, body, 0, unroll=False)

        wait(g_a())
        start(w_copies(last, srows_a, trows_a, oss_a, ost_a))
        wait(w_copies(last, srows_a, trows_a, oss_a, ost_a))

    return gather_kernel


def kernel(x_gat_fin, edge_index):
    V, D = x_gat_fin.shape
    E = edge_index.shape[1]
    idx = edge_index.astype(jnp.int32).reshape(-1)
    return _build_gather(V, D, E, 80)(x_gat_fin, idx)


# speedup vs baseline: 1.3969x; 1.3969x over previous
import functools

import jax
import jax.numpy as jnp
from jax import lax
from jax.experimental import pallas as pl
from jax.experimental.pallas import tpu as pltpu
from jax.experimental.pallas import tpu_sc as plsc


@functools.lru_cache(maxsize=None)
def _build_gather(V, D, E, chunk):
    info = plsc.get_sparse_core_info()
    NC, NS = info.num_cores, info.num_subcores
    NW = NC * NS
    assert E % NW == 0
    e_per_w = E // NW
    assert e_per_w % chunk == 0 and chunk % 8 == 0
    nchunks = e_per_w // chunk
    assert nchunks % 2 == 1 and nchunks >= 3
    npairs = nchunks // 2
    last = nchunks - 1
    v_per_s = (V // NS) // 8 * 8
    v_last = V - v_per_s * (NS - 1)
    mesh = plsc.VectorSubcoreMesh(core_axis_name="c", subcore_axis_name="s")

    @functools.partial(
        pl.kernel,
        mesh=mesh,
        out_type=jax.ShapeDtypeStruct((E, 2 * D), jnp.float32),
        scratch_types=[
            pltpu.VMEM_SHARED((V, D), jnp.float32),
            pltpu.VMEM((chunk,), jnp.int32),
            pltpu.VMEM((chunk,), jnp.int32),
            pltpu.VMEM((chunk,), jnp.int32),
            pltpu.VMEM((chunk,), jnp.int32),
            pltpu.VMEM((chunk, D), jnp.float32),
            pltpu.VMEM((chunk, D), jnp.float32),
            pltpu.VMEM((chunk, D), jnp.float32),
            pltpu.VMEM((chunk, D), jnp.float32),
            pltpu.SemaphoreType.DMA,
            pltpu.SemaphoreType.DMA,
            pltpu.SemaphoreType.DMA,
            pltpu.SemaphoreType.DMA,
            pltpu.SemaphoreType.DMA,
            pltpu.SemaphoreType.DMA,
            pltpu.SemaphoreType.DMA,
            pltpu.SemaphoreType.DMA,
            pltpu.SemaphoreType.DMA,
            pltpu.SemaphoreType.DMA,
            pltpu.SemaphoreType.DMA,
            pltpu.SemaphoreType.DMA,
        ],
    )
    def gather_kernel(
        table_hbm, idx_hbm, out_hbm,
        table_s, sidx_a, tidx_a, sidx_b, tidx_b,
        srows_a, trows_a, srows_b, trows_b,
        iss_a, ist_a, iss_b, ist_b,
        gss_a, gst_a, gss_b, gst_b,
        oss_a, ost_a, oss_b, ost_b,
    ):
        sub = lax.axis_index("s")
        wid = sub * NC + lax.axis_index("c")
        base = wid * e_per_w

        row0 = sub * v_per_s
        nrow = jnp.where(sub == NS - 1, v_last, v_per_s)
        pltpu.sync_copy(
            table_hbm.at[pl.ds(row0, nrow)], table_s.at[pl.ds(row0, nrow)]
        )
        plsc.subcore_barrier()

        def i_copies(g, sbuf, tbuf, ssem, tsem):
            off = base + g * chunk
            return (
                pltpu.make_async_copy(
                    idx_hbm.at[pl.ds(off, chunk)], sbuf, ssem
                ),
                pltpu.make_async_copy(
                    idx_hbm.at[pl.ds(E + off, chunk)], tbuf, tsem
                ),
            )

        def g_copies(sibuf, tibuf, sbuf, tbuf, ssem, tsem):
            return (
                pltpu.make_async_copy(table_s.at[sibuf], sbuf, ssem),
                pltpu.make_async_copy(table_s.at[tibuf], tbuf, tsem),
            )

        def w_copies(g, sbuf, tbuf, ssem, tsem):
            orow = base + g * chunk
            return (
                pltpu.make_async_copy(
                    sbuf, out_hbm.at[pl.ds(orow, chunk), pl.ds(0, D)], ssem
                ),
                pltpu.make_async_copy(
                    tbuf, out_hbm.at[pl.ds(orow, chunk), pl.ds(D, D)], tsem
                ),
            )

        def g_a():
            return g_copies(sidx_a, tidx_a, srows_a, trows_a, gss_a, gst_a)

        def g_b():
            return g_copies(sidx_b, tidx_b, srows_b, trows_b, gss_b, gst_b)

        def i_a(g):
            return i_copies(g, sidx_a, tidx_a, iss_a, ist_a)

        def i_b(g):
            return i_copies(g, sidx_b, tidx_b, iss_b, ist_b)

        def start(copies):
            for c in copies:
                c.start()

        def wait(copies):
            for c in copies:
                c.wait()

        start(i_a(0))
        start(i_b(1))
        wait(i_a(0))
        start(g_a())

        def body(p, carry):
            g0 = 2 * p
            g1 = g0 + 1
            wait(i_b(g1))
            start(g_b())
            wait(g_a())
            start(i_a(g0 + 2))
            start(w_copies(g0, srows_a, trows_a, oss_a, ost_a))
            wait(g_b())

            @pl.when(p < npairs - 1)
            def _():
                start(i_b(g1 + 2))

            start(w_copies(g1, srows_b, trows_b, oss_b, ost_b))
            wait(w_copies(g0, srows_a, trows_a, oss_a, ost_a))
            wait(i_a(g0 + 2))
            start(g_a())
            wait(w_copies(g1, srows_b, trows_b, oss_b, ost_b))
            return carry

        lax.fori_loop(0, npairs, body, 0, unroll=False)

        wait(g_a())
        start(w_copies(last, srows_a, trows_a, oss_a, ost_a))
        wait(w_copies(last, srows_a, trows_a, oss_a, ost_a))

    return gather_kernel


def kernel(x_gat_fin, edge_index):
    V, D = x_gat_fin.shape
    E = edge_index.shape[1]
    idx = edge_index.astype(jnp.int32).reshape(-1)
    return _build_gather(V, D, E, 80)(x_gat_fin, idx)


# speedup vs baseline: 1.4325x; 1.0255x over previous
import functools

import jax
import jax.numpy as jnp
from jax import lax
from jax.experimental import pallas as pl
from jax.experimental.pallas import tpu as pltpu
from jax.experimental.pallas import tpu_sc as plsc

_NSETS = 4


@functools.lru_cache(maxsize=None)
def _build_gather(V, D, E, chunk):
    info = plsc.get_sparse_core_info()
    NC, NS = info.num_cores, info.num_subcores
    NW = NC * NS
    assert E % NW == 0
    e_per_w = E // NW
    assert e_per_w % chunk == 0 and chunk % 8 == 0
    nchunks = e_per_w // chunk
    assert nchunks > 2 * _NSETS
    nrounds = (nchunks + _NSETS - 1) // _NSETS
    v_per_s = (V // NS) // 8 * 8
    v_last = V - v_per_s * (NS - 1)
    mesh = plsc.VectorSubcoreMesh(core_axis_name="c", subcore_axis_name="s")

    scratch = [pltpu.VMEM_SHARED((V, D), jnp.float32)]
    scratch += [pltpu.VMEM((chunk,), jnp.int32) for _ in range(2 * _NSETS)]
    scratch += [pltpu.VMEM((chunk, D), jnp.float32) for _ in range(2 * _NSETS)]
    scratch += [pltpu.SemaphoreType.DMA for _ in range(6 * _NSETS)]

    @functools.partial(
        pl.kernel,
        mesh=mesh,
        out_type=jax.ShapeDtypeStruct((E, 2 * D), jnp.float32),
        scratch_types=scratch,
    )
    def gather_kernel(table_hbm, idx_hbm, out_hbm, table_s, *bufs):
        sidx = bufs[0:2 * _NSETS:2]
        tidx = bufs[1:2 * _NSETS:2]
        srows = bufs[2 * _NSETS:4 * _NSETS:2]
        trows = bufs[2 * _NSETS + 1:4 * _NSETS:2]
        sems = bufs[4 * _NSETS:]
        isem = (sems[0:2 * _NSETS:2], sems[1:2 * _NSETS:2])
        gsem = (sems[2 * _NSETS:4 * _NSETS:2], sems[2 * _NSETS + 1:4 * _NSETS:2])
        wsem = (sems[4 * _NSETS:6 * _NSETS:2], sems[4 * _NSETS + 1:6 * _NSETS:2])

        sub = lax.axis_index("s")
        wid = sub * NC + lax.axis_index("c")
        base = wid * e_per_w

        row0 = sub * v_per_s
        nrow = jnp.where(sub == NS - 1, v_last, v_per_s)
        pltpu.sync_copy(
            table_hbm.at[pl.ds(row0, nrow)], table_s.at[pl.ds(row0, nrow)]
        )
        plsc.subcore_barrier()

        def i_copies(g, k):
            off = base + g * chunk
            return (
                pltpu.make_async_copy(
                    idx_hbm.at[pl.ds(off, chunk)], sidx[k], isem[0][k]
                ),
                pltpu.make_async_copy(
                    idx_hbm.at[pl.ds(E + off, chunk)], tidx[k], isem[1][k]
                ),
            )

        def g_copies(k):
            return (
                pltpu.make_async_copy(table_s.at[sidx[k]], srows[k], gsem[0][k]),
                pltpu.make_async_copy(table_s.at[tidx[k]], trows[k], gsem[1][k]),
            )

        def w_copies(g, k):
            orow = base + g * chunk
            return (
                pltpu.make_async_copy(
                    srows[k], out_hbm.at[pl.ds(orow, chunk), pl.ds(0, D)],
                    wsem[0][k],
                ),
                pltpu.make_async_copy(
                    trows[k], out_hbm.at[pl.ds(orow, chunk), pl.ds(D, D)],
                    wsem[1][k],
                ),
            )

        def start(copies):
            for c in copies:
                c.start()

        def wait(copies):
            for c in copies:
                c.wait()

        for k in range(_NSETS):
            start(i_copies(k, k))
        wait(i_copies(0, 0))
        start(g_copies(0))
        wait(i_copies(1, 1))
        start(g_copies(1))

        def slot(s, k):
            @pl.when(s < nchunks)
            def _():
                wait(g_copies(k))
                start(w_copies(s, k))

                @pl.when(s + _NSETS < nchunks)
                def _():
                    start(i_copies(s + _NSETS, k))

                @pl.when(s + 2 < nchunks)
                def _():
                    k2 = (k + 2) % _NSETS
                    wait(i_copies(s + 2, k2))

                    @pl.when(s >= 2)
                    def _():
                        wait(w_copies(s - 2, k2))

                    start(g_copies(k2))

        def body(r, carry):
            s0 = r * _NSETS
            for k in range(_NSETS):
                slot(s0 + k, k)
            return carry

        lax.fori_loop(0, nrounds, body, 0, unroll=False)

        for g in range(nchunks - 4, nchunks):
            wait(w_copies(g, g % _NSETS))

    return gather_kernel


def kernel(x_gat_fin, edge_index):
    V, D = x_gat_fin.shape
    E = edge_index.shape[1]
    idx = edge_index.astype(jnp.int32).reshape(-1)
    return _build_gather(V, D, E, 40)(x_gat_fin, idx)
